# baseline (speedup 1.0000x reference)
import functools

import jax
import jax.numpy as jnp
from jax import lax
from jax.experimental import pallas as pl
from jax.experimental.pallas import tpu as pltpu
from jax.experimental.pallas import tpu_sc as plsc

_B = 16384
_D = 64
_C = 100000
_CFULL = 99968
_NGRP = 781
_SINK = _B

_info = plsc.get_sparse_core_info()
_NC = _info.num_cores
_NS = _info.num_subcores
_L = _info.num_lanes
_NW = _NC * _NS
_BPW = _B // _NW
_CW = 512
_NCH = 13
_NRING = 8

_mesh = plsc.VectorSubcoreMesh(core_axis_name="c", subcore_axis_name="s")
_params = pltpu.CompilerParams(needs_layout_passes=False)


@functools.partial(
    pl.kernel,
    mesh=_mesh,
    compiler_params=_params,
    out_type=jax.ShapeDtypeStruct((_B + 8, 2 * _D), jnp.float32),
    scratch_types=[
        pltpu.VMEM((_B,), jnp.int32),
        pltpu.VMEM((_B + 32,), jnp.int32),
        pltpu.VMEM((_B + 32,), jnp.int32),
        pltpu.VMEM((_D, _CW), jnp.float32),
        pltpu.VMEM((_NRING, _L, 2 * _D), jnp.float32),
        pltpu.VMEM((_NRING, _L), jnp.int32),
        pltpu.VMEM((_D, 32), jnp.float32),
        pltpu.SemaphoreType.DMA((_NRING,)),
    ],
)
def _route_centers(labels_hbm, ct_hbm, tail_hbm, out_hbm,
                   lbl_v, b_v, b2_v, t_v, rs_v, pidx_v, tt_v, ssem):
    wid = lax.axis_index("s") * _NC + lax.axis_index("c")
    g0 = (_NGRP * wid) // _NW
    g1 = (_NGRP * (wid + 1)) // _NW
    c_lo = g0 * 128
    c_hi = g1 * 128
    is_last = wid == _NW - 1

    pltpu.sync_copy(labels_hbm, lbl_v)

    iota = lax.iota(jnp.int32, _L)
    iotas = [kk * _L + lax.iota(jnp.int32, _L) for kk in range(_D // _L)]

    def _pc(mask):
        p = plsc.all_reduce_population_count(mask)
        return p[0] if getattr(p, "ndim", 0) else p

    def scan_body(j, cnt):
        lv = lbl_v[pl.ds(j * _L, _L)]
        m = (lv >= c_lo) & (lv < c_hi)
        m = m | ((lv >= _CFULL) & is_last)
        plsc.store_compressed(b_v.at[pl.ds(cnt, _L)], j * _L + iota, mask=m)
        return cnt + _pc(m)

    cnt = lax.fori_loop(0, _B // _L, scan_body, jnp.int32(0))
    b_v[pl.ds(cnt, _L)] = jnp.zeros((_L,), jnp.int32)
    nq = (cnt + _L - 1) // _L

    def hist_body(q, h):
        bq = b_v[pl.ds(q * _L, _L)]
        cq = plsc.load_gather(lbl_v, [bq & (_B - 1)])
        valid = (q * _L + iota) < cnt
        main = valid & (cq < c_hi)
        bkt = (cq - c_lo) >> 9
        h = list(h)
        for ck in range(_NCH):
            h[ck] = h[ck] + _pc(main & (bkt == ck))
        h[_NCH] = h[_NCH] + _pc(valid & (cq >= _CFULL))
        return tuple(h)

    hist = lax.fori_loop(0, nq, hist_body,
                         tuple(jnp.int32(0) for _ in range(_NCH + 1)))
    offs = []
    acc = jnp.int32(0)
    for ck in range(_NCH + 1):
        offs.append(acc)
        acc = acc + hist[ck]

    def place_body(q, cur):
        bq = b_v[pl.ds(q * _L, _L)]
        cq = plsc.load_gather(lbl_v, [bq & (_B - 1)])
        valid = (q * _L + iota) < cnt
        main = valid & (cq < c_hi)
        bkt = (cq - c_lo) >> 9
        cur = list(cur)
        for ck in range(_NCH):
            m = main & (bkt == ck)
            plsc.store_compressed(b2_v.at[pl.ds(cur[ck], _L)], bq, mask=m)
            cur[ck] = cur[ck] + _pc(m)
        mt = valid & (cq >= _CFULL)
        plsc.store_compressed(b2_v.at[pl.ds(cur[_NCH], _L)], bq, mask=mt)
        cur[_NCH] = cur[_NCH] + _pc(mt)
        return tuple(cur)

    lax.fori_loop(0, nq, place_body, tuple(offs))

    def drain_slot(slot):
        pltpu.make_async_copy(out_hbm.at[pl.ds(0, _L)],
                              rs_v.at[0], ssem.at[slot]).wait()

    def fire(slot):
        pltpu.async_copy(rs_v.at[slot], out_hbm.at[pidx_v.at[slot]],
                         ssem.at[slot])

    nw = jnp.int32(0)
    for ck in range(_NCH):
        seg_o = offs[ck]
        seg_n = hist[ck]

        @pl.when(seg_n > 0)
        def _(ck=ck, seg_o=seg_o, seg_n=seg_n):
            wlo = c_lo + ck * _CW
            sl0 = jnp.minimum(wlo, _CFULL - _CW)
            pltpu.sync_copy(ct_hbm.at[:, pl.ds(sl0, _CW)], t_v)

        def emit_body(q, nw, ck=ck, seg_o=seg_o, seg_n=seg_n):
            wlo = c_lo + ck * _CW
            sl0 = jnp.minimum(wlo, _CFULL - _CW)
            base = seg_o + q * _L
            bq = b2_v[pl.ds(base, _L)] & (_B - 1)
            cq = plsc.load_gather(lbl_v, [bq])
            valid = (q * _L + iota) < seg_n
            colq = jnp.minimum(jnp.maximum(cq - sl0, 0), _CW - 1)
            slot = nw % _NRING

            @pl.when(nw >= _NRING)
            def _():
                drain_slot(slot)

            for i in range(_L):
                col = jnp.broadcast_to(colq[i], (_L,))
                for kk in range(_D // _L):
                    v = plsc.load_gather(t_v, [iotas[kk], col])
                    rs_v[slot, i, pl.ds(kk * _L, _L)] = v
            pidx_v[slot, :] = jnp.where(valid, bq, _SINK)
            fire(slot)
            return nw + 1

        nw = lax.fori_loop(0, (seg_n + _L - 1) // _L, emit_body, nw)

    seg_o = offs[_NCH]
    seg_n = hist[_NCH]

    @pl.when(seg_n > 0)
    def _():
        pltpu.sync_copy(tail_hbm, tt_v)

    def tail_body(q, nw):
        base = seg_o + q * _L
        bq = b2_v[pl.ds(base, _L)] & (_B - 1)
        cq = plsc.load_gather(lbl_v, [bq])
        valid = (q * _L + iota) < seg_n
        colq = jnp.minimum(jnp.maximum(cq - _CFULL, 0), 31)
        slot = nw % _NRING

        @pl.when(nw >= _NRING)
        def _():
            drain_slot(slot)

        for i in range(_L):
            col = jnp.broadcast_to(colq[i], (_L,))
            for kk in range(_D // _L):
                v = plsc.load_gather(tt_v, [iotas[kk], col])
                rs_v[slot, i, pl.ds(kk * _L, _L)] = v
        pidx_v[slot, :] = jnp.where(valid, bq, _SINK)
        fire(slot)
        return nw + 1

    nw = lax.fori_loop(0, (seg_n + _L - 1) // _L, tail_body, nw)

    for s in range(_NRING):
        @pl.when(s < nw)
        def _(s=s):
            drain_slot(s)


@functools.partial(
    pl.kernel,
    mesh=_mesh,
    compiler_params=_params,
    out_type=jax.ShapeDtypeStruct((_NW * _L,), jnp.float32),
    scratch_types=[
        pltpu.VMEM((_D, _BPW), jnp.float32),
        pltpu.VMEM((_BPW, 2 * _D), jnp.float32),
        pltpu.VMEM((_L,), jnp.float32),
        pltpu.SemaphoreType.DMA,
        pltpu.SemaphoreType.DMA,
    ],
)
def _center_loss_partials(xt_hbm, cg_hbm, out_hbm, x_v, c_v, tot_v,
                          xsem, csem):
    wid = lax.axis_index("s") * _NC + lax.axis_index("c")
    base = wid * _BPW

    xcopy = pltpu.async_copy(xt_hbm.at[:, pl.ds(base, _BPW)], x_v, xsem)
    ccopy = pltpu.async_copy(cg_hbm.at[pl.ds(base, _BPW)], c_v, csem)
    xcopy.wait()
    ccopy.wait()

    iotas = [kk * _L + lax.iota(jnp.int32, _L) for kk in range(_D // _L)]

    def row_body(r, tot):
        col = jnp.broadcast_to(r, (_L,)).astype(jnp.int32)
        acc = jnp.zeros((_L,), jnp.float32)
        for kk in range(_D // _L):
            xa = plsc.load_gather(x_v, [iotas[kk], col])
            ca = c_v[r, pl.ds(kk * _L, _L)]
            dd = xa - ca
            acc = acc + dd * dd
        dist = jnp.sum(acc)
        dist = jnp.minimum(jnp.maximum(dist, 1e-12), 1e12)
        return tot + dist

    tot = lax.fori_loop(0, _BPW, row_body, jnp.float32(0.0))
    iota = lax.iota(jnp.int32, _L)
    tot_v[...] = jnp.where(iota < 1, tot, jnp.float32(0.0))
    pltpu.sync_copy(tot_v, out_hbm.at[pl.ds(wid * _L, _L)])


def kernel(x, labels, centers):
    labels32 = labels.astype(jnp.int32)
    centers_t = centers.T
    tail_t = centers[_CFULL:].T
    staged = _route_centers(labels32, centers_t, tail_t)
    partials = _center_loss_partials(x.T, staged)
    return jnp.sum(partials) * (1.0 / _B)


# speedup vs baseline: 1.9202x; 1.9202x over previous
import functools

import jax
import jax.numpy as jnp
from jax import lax
from jax.experimental import pallas as pl
from jax.experimental.pallas import tpu as pltpu
from jax.experimental.pallas import tpu_sc as plsc

_B = 16384
_D = 64
_C = 100000

_info = plsc.get_sparse_core_info()
_NC = _info.num_cores
_NS = _info.num_subcores
_L = _info.num_lanes
_NW = _NC * _NS
_BPW = _B // _NW
_CH = 32
_NPAIR = _BPW // (2 * _CH)

_mesh = plsc.VectorSubcoreMesh(core_axis_name="c", subcore_axis_name="s")


@functools.partial(
    pl.kernel,
    mesh=_mesh,
    compiler_params=pltpu.CompilerParams(needs_layout_passes=False),
    out_type=jax.ShapeDtypeStruct((_NW * _L,), jnp.float32),
    scratch_types=[
        pltpu.VMEM((_BPW,), jnp.int32),
        pltpu.VMEM((_D, _BPW), jnp.float32),
        pltpu.VMEM((2, _CH, 8, _D), jnp.float32),
        pltpu.VMEM((_L,), jnp.float32),
        pltpu.SemaphoreType.DMA,
        pltpu.SemaphoreType.DMA,
        pltpu.SemaphoreType.DMA,
    ],
)
def _center_loss_partials(xt_hbm, labels_hbm, centers_hbm, out_hbm,
                          lbl_v, x_v, c_v, tot_v, semA, semB, xsem):
    wid = lax.axis_index("s") * _NC + lax.axis_index("c")
    base = wid * _BPW

    pltpu.sync_copy(labels_hbm.at[pl.ds(base, _BPW)], lbl_v)
    xcopy = pltpu.async_copy(xt_hbm.at[:, pl.ds(base, _BPW)], x_v, xsem)

    def fire(k, buf, sem):
        for g in range(_CH // _L):
            lblv = lbl_v[pl.ds(k * _CH + g * _L, _L)]
            tv = lblv >> 3
            for i in range(_L):
                pltpu.async_copy(centers_hbm.at[tv[i]],
                                 buf.at[g * _L + i], sem)

    def drain(buf, sem):
        pltpu.make_async_copy(centers_hbm.at[pl.ds(0, _CH)], buf, sem).wait()

    iotas = [kk * _L + lax.iota(jnp.int32, _L) for kk in range(_D // _L)]

    def comp(k, buf, tot):
        for g in range(_CH // _L):
            r0 = k * _CH + g * _L
            lblv = lbl_v[pl.ds(r0, _L)]
            sv = lblv & 7
            for i in range(_L):
                col = jnp.broadcast_to(r0 + i, (_L,)).astype(jnp.int32)
                acc = jnp.zeros((_L,), jnp.float32)
                for kk in range(_D // _L):
                    xa = plsc.load_gather(x_v, [iotas[kk], col])
                    ca = buf[g * _L + i, sv[i], pl.ds(kk * _L, _L)]
                    dd = xa - ca
                    acc = acc + dd * dd
                dist = jnp.sum(acc)
                dist = jnp.minimum(jnp.maximum(dist, 1e-12), 1e12)
                tot = tot + dist
        return tot

    fire(0, c_v.at[0], semA)
    xcopy.wait()

    def pair_body(m, tot):
        fire(2 * m + 1, c_v.at[1], semB)
        drain(c_v.at[0], semA)
        tot = comp(2 * m, c_v.at[0], tot)

        @pl.when(m < _NPAIR - 1)
        def _():
            fire(2 * m + 2, c_v.at[0], semA)

        drain(c_v.at[1], semB)
        return comp(2 * m + 1, c_v.at[1], tot)

    tot = lax.fori_loop(0, _NPAIR, pair_body, jnp.float32(0.0))
    iota = lax.iota(jnp.int32, _L)
    tot_v[...] = jnp.where(iota < 1, tot, jnp.float32(0.0))
    pltpu.sync_copy(tot_v, out_hbm.at[pl.ds(wid * _L, _L)])


def kernel(x, labels, centers):
    centers3 = centers.reshape(_C // 8, 8, _D)
    partials = _center_loss_partials(x.T, labels.astype(jnp.int32), centers3)
    return jnp.sum(partials) * (1.0 / _B)


# speedup vs baseline: 1.9930x; 1.0379x over previous
import functools

import jax
import jax.numpy as jnp
from jax import lax
from jax.experimental import pallas as pl
from jax.experimental.pallas import tpu as pltpu
from jax.experimental.pallas import tpu_sc as plsc

_B = 16384
_D = 64
_C = 100000

_info = plsc.get_sparse_core_info()
_NC = _info.num_cores
_NS = _info.num_subcores
_L = _info.num_lanes
_NW = _NC * _NS
_BPW = _B // _NW
_CH = 32
_NPAIR = _BPW // (2 * _CH)

_mesh = plsc.VectorSubcoreMesh(core_axis_name="c", subcore_axis_name="s")


@functools.partial(
    pl.kernel,
    mesh=_mesh,
    compiler_params=pltpu.CompilerParams(needs_layout_passes=False),
    out_type=jax.ShapeDtypeStruct((_NW * _L,), jnp.float32),
    scratch_types=[
        pltpu.VMEM((_BPW,), jnp.int32),
        pltpu.VMEM((_D, _BPW), jnp.float32),
        pltpu.VMEM((2, _CH, 8, _D), jnp.float32),
        pltpu.VMEM((_L,), jnp.float32),
        pltpu.SemaphoreType.DMA,
        pltpu.SemaphoreType.DMA,
        pltpu.SemaphoreType.DMA,
    ],
)
def _center_loss_partials(xt_hbm, labels_hbm, centers_hbm, out_hbm,
                          lbl_v, x_v, c_v, tot_v, semA, semB, xsem):
    wid = lax.axis_index("s") * _NC + lax.axis_index("c")
    base = wid * _BPW

    pltpu.sync_copy(labels_hbm.at[pl.ds(base, _BPW)], lbl_v)
    xcopy = pltpu.async_copy(xt_hbm.at[:, pl.ds(base, _BPW)], x_v, xsem)

    def fire(k, buf, sem):
        for g in range(_CH // _L):
            lblv = lbl_v[pl.ds(k * _CH + g * _L, _L)]
            tv = lblv >> 3
            for i in range(_L):
                pltpu.async_copy(centers_hbm.at[tv[i]],
                                 buf.at[g * _L + i], sem)

    def drain(buf, sem):
        pltpu.make_async_copy(centers_hbm.at[pl.ds(0, _CH)], buf, sem).wait()

    iota = lax.iota(jnp.int32, _L)

    def comp(k, buf, tot):
        for g in range(_CH // _L):
            r0 = k * _CH + g * _L
            lblv = lbl_v[pl.ds(r0, _L)]
            subv = lblv & 7
            slotv = g * _L + iota
            acc = jnp.zeros((_L,), jnp.float32)
            for f in range(_D):
                xa = x_v[f, pl.ds(r0, _L)]
                ca = plsc.load_gather(
                    buf, [slotv, subv, jnp.full((_L,), f, jnp.int32)])
                dd = xa - ca
                acc = acc + dd * dd
            acc = jnp.minimum(jnp.maximum(acc, 1e-12), 1e12)
            tot = tot + acc
        return tot

    fire(0, c_v.at[0], semA)
    xcopy.wait()

    def pair_body(m, tot):
        fire(2 * m + 1, c_v.at[1], semB)
        drain(c_v.at[0], semA)
        tot = comp(2 * m, c_v.at[0], tot)

        @pl.when(m < _NPAIR - 1)
        def _():
            fire(2 * m + 2, c_v.at[0], semA)

        drain(c_v.at[1], semB)
        return comp(2 * m + 1, c_v.at[1], tot)

    tot = lax.fori_loop(0, _NPAIR, pair_body,
                        jnp.zeros((_L,), jnp.float32))
    tot_v[...] = tot
    pltpu.sync_copy(tot_v, out_hbm.at[pl.ds(wid * _L, _L)])


def kernel(x, labels, centers):
    centers3 = centers.reshape(_C // 8, 8, _D)
    partials = _center_loss_partials(x.T, labels.astype(jnp.int32), centers3)
    return jnp.sum(partials) * (1.0 / _B)
